# Initial kernel scaffold; baseline (speedup 1.0000x reference)
import jax
import jax.numpy as jnp
from jax.experimental import pallas as pl


def kernel(x, gate_w, gate_weights, up_weights, down_weights):
    raise NotImplementedError("write your pallas kernel here")


# speedup vs baseline: 5.2246x; 5.2246x over previous
import functools

import jax
import jax.numpy as jnp
from jax import lax
from jax.experimental import pallas as pl
from jax.experimental.pallas import tpu as pltpu
from jax.experimental.pallas import tpu_sc as plsc

T = 2048
E = 64
D = 1024
F = 1024
BM = 64
PT = 6144
NT = PT // BM
EPW = 4
LROW = T + BM
GCH = 64
TRASH_TOK = T
TRASH_TILE = 96

_mesh = plsc.VectorSubcoreMesh(core_axis_name="c", subcore_axis_name="s")
_CP = pltpu.CompilerParams(needs_layout_passes=False)
_i16 = functools.partial(lax.broadcasted_iota, jnp.int32, (16,), 0)


def _router_body(x_ref, gw_ref, idx_ref, prob_ref):
    x = x_ref[...]
    gw = gw_ref[...]
    logits = lax.dot_general(x, gw, (((1,), (1,)), ((), ())),
                             preferred_element_type=jnp.float32)
    m = jnp.max(logits, axis=1, keepdims=True)
    z = jnp.sum(jnp.exp(logits - m), axis=1, keepdims=True)
    ii = lax.broadcasted_iota(jnp.int32, logits.shape, 1)
    idx_ref[...] = jnp.min(jnp.where(logits == m, ii, E), axis=1, keepdims=True)
    prob_ref[...] = 1.0 / z


def _router(x, gate_w):
    return pl.pallas_call(
        _router_body,
        out_shape=(
            jax.ShapeDtypeStruct((T, 1), jnp.int32),
            jax.ShapeDtypeStruct((T, 1), jnp.float32),
        ),
    )(x, gate_w)


@functools.partial(
    pl.kernel,
    out_type=(
        jax.ShapeDtypeStruct((16, 16), jnp.int32),
        jax.ShapeDtypeStruct((E, LROW), jnp.int32),
        jax.ShapeDtypeStruct((E, LROW), jnp.float32),
    ),
    mesh=_mesh,
    compiler_params=_CP,
    scratch_types=[
        pltpu.VMEM((T,), jnp.int32),
        pltpu.VMEM((T,), jnp.float32),
        [pltpu.VMEM((LROW,), jnp.int32)] * EPW,
        [pltpu.VMEM((LROW,), jnp.float32)] * EPW,
        pltpu.VMEM((16,), jnp.int32),
    ],
)
def _count(idx_hbm, prob_hbm, cnt_hbm, tokscr_hbm, pscr_hbm,
           idx_v, prob_v, toklist, problist, stage):
    c = lax.axis_index("c")
    s = lax.axis_index("s")
    iota16 = _i16()

    @pl.when(c == 0)
    def _work():
        pltpu.sync_copy(idx_hbm, idx_v)
        pltpu.sync_copy(prob_hbm, prob_v)
        cv = jnp.zeros((16,), jnp.int32)
        for k in range(EPW):
            e = s * EPW + k

            def _scan(j, ptr, k=k, e=e):
                v = idx_v[pl.ds(j * 16, 16)]
                mk = v == e
                plsc.store_compressed(toklist[k].at[pl.ds(ptr, 16)],
                                      iota16 + j * 16, mask=mk)
                plsc.store_compressed(problist[k].at[pl.ds(ptr, 16)],
                                      prob_v[pl.ds(j * 16, 16)], mask=mk)
                return ptr + jnp.sum(mk.astype(jnp.int32))

            cnt = lax.fori_loop(0, T // 16, _scan, jnp.int32(0))
            cv = jnp.where(iota16 == k, cnt, cv)
            pltpu.sync_copy(toklist[k], tokscr_hbm.at[e])
            pltpu.sync_copy(problist[k], pscr_hbm.at[e])
        stage[...] = cv
        pltpu.sync_copy(stage, cnt_hbm.at[s])


@functools.partial(
    pl.kernel,
    out_type=(
        jax.ShapeDtypeStruct((PT,), jnp.int32),
        jax.ShapeDtypeStruct((PT,), jnp.float32),
        jax.ShapeDtypeStruct((T + 32,), jnp.int32),
        jax.ShapeDtypeStruct((128,), jnp.int32),
        jax.ShapeDtypeStruct((16,), jnp.int32),
    ),
    mesh=_mesh,
    compiler_params=_CP,
    scratch_types=[
        pltpu.VMEM((16, 16), jnp.int32),
        [pltpu.VMEM((LROW,), jnp.int32)] * EPW,
        [pltpu.VMEM((LROW,), jnp.float32)] * EPW,
        pltpu.VMEM((32,), jnp.int32),
        pltpu.VMEM((32,), jnp.int32),
        pltpu.VMEM((16,), jnp.int32),
        pltpu.VMEM((16,), jnp.int32),
        pltpu.VMEM((16,), jnp.int32),
        pltpu.SemaphoreType.DMA,
    ],
)
def _place(cnt_hbm, tokscr_hbm, pscr_hbm, tok_hbm, pslot_hbm, slot_hbm,
           tile_hbm, nt_hbm, allcnt, toklist, problist, idxch, valch,
           idxt, valt, stage, sem):
    c = lax.axis_index("c")
    s = lax.axis_index("s")
    iota16 = _i16()

    @pl.when(c == 0)
    def _work():
        pltpu.sync_copy(cnt_hbm, allcnt)
        acc = jnp.int32(0)
        starts = [jnp.int32(0)] * EPW
        ntls = [jnp.int32(0)] * EPW
        cnts = [jnp.int32(0)] * EPW
        for ww in range(16):
            row = allcnt[ww]
            for k in range(EPW):
                ck = jnp.sum(jnp.where(iota16 == k, row, 0))
                ntl = (ck + BM - 1) >> 6
                starts[k] = jnp.where(s == ww, acc, starts[k])
                ntls[k] = jnp.where(s == ww, ntl, ntls[k])
                cnts[k] = jnp.where(s == ww, ck, cnts[k])
                acc = acc + ntl
        total_tiles = acc

        for k in range(EPW):
            e = s * EPW + k
            st_slot = starts[k] * BM
            cnt = cnts[k]
            pltpu.sync_copy(tokscr_hbm.at[e], toklist[k])
            pltpu.sync_copy(pscr_hbm.at[e], problist[k])
            for q in range(BM // 16):
                toklist[k][pl.ds(cnt + q * 16, 16)] = jnp.zeros((16,), jnp.int32)
                problist[k][pl.ds(cnt + q * 16, 16)] = jnp.zeros((16,), jnp.float32)

            def _fire(j, _, k=k, st_slot=st_slot):
                pltpu.async_copy(toklist[k].at[pl.ds(j * BM, BM)],
                                 tok_hbm.at[pl.ds(st_slot + j * BM, BM)], sem)
                pltpu.async_copy(problist[k].at[pl.ds(j * BM, BM)],
                                 pslot_hbm.at[pl.ds(st_slot + j * BM, BM)], sem)
                return 0

            lax.fori_loop(0, ntls[k], _fire, 0)

            def _sct(j, _, k=k, st_slot=st_slot, cnt=cnt):
                for q in range(2):
                    v = toklist[k][pl.ds(j * 32 + q * 16, 16)]
                    pos = j * 32 + q * 16 + iota16
                    idxch[pl.ds(q * 16, 16)] = jnp.where(pos < cnt, v, TRASH_TOK)
                    valch[pl.ds(q * 16, 16)] = st_slot + pos
                pltpu.sync_copy(valch, slot_hbm.at[idxch])
                return 0

            lax.fori_loop(0, (cnt + 31) >> 5, _sct, 0)

            for ch in range(2):
                lane = ch * 16 + iota16
                idxt[...] = jnp.where(lane < ntls[k], starts[k] + lane, TRASH_TILE)
                valt[...] = jnp.zeros((16,), jnp.int32) + e
                pltpu.sync_copy(valt, tile_hbm.at[idxt])

        for k in range(EPW):
            st_slot = starts[k] * BM

            def _drain(j, _, k=k, st_slot=st_slot):
                pltpu.make_async_copy(
                    toklist[k].at[pl.ds(j * BM, BM)],
                    tok_hbm.at[pl.ds(st_slot + j * BM, BM)], sem).wait()
                pltpu.make_async_copy(
                    problist[k].at[pl.ds(j * BM, BM)],
                    pslot_hbm.at[pl.ds(st_slot + j * BM, BM)], sem).wait()
                return 0

            lax.fori_loop(0, ntls[k], _drain, 0)

        @pl.when(s == 0)
        def _emit():
            stage[...] = jnp.where(iota16 == 0, total_tiles, 0)
            pltpu.sync_copy(stage, nt_hbm)


@functools.partial(
    pl.kernel,
    out_type=jax.ShapeDtypeStruct((PT, D), jnp.float32),
    mesh=_mesh,
    compiler_params=_CP,
    scratch_types=[
        pltpu.VMEM((GCH,), jnp.int32),
        pltpu.VMEM((GCH, D), jnp.float32),
        pltpu.VMEM((16,), jnp.int32),
        pltpu.SemaphoreType.DMA,
    ],
)
def _gather(x_hbm, tok_hbm, nt_hbm, xs_hbm, idxb, rows, ntv, sem):
    c = lax.axis_index("c")
    s = lax.axis_index("s")
    wid = s * 2 + c
    iota16 = _i16()
    pltpu.sync_copy(nt_hbm, ntv)
    total = jnp.sum(jnp.where(iota16 == 0, ntv[...], 0)) * BM
    for cc in range(PT // 32 // GCH):
        base = wid * (PT // 32) + cc * GCH

        @pl.when(base < total)
        def _chunk(base=base):
            pltpu.sync_copy(tok_hbm.at[pl.ds(base, GCH)], idxb)
            for q in range(GCH // 16):
                v = idxb[pl.ds(q * 16, 16)]
                pos = base + q * 16 + iota16
                idxb[pl.ds(q * 16, 16)] = jnp.where(pos < total, v, 0)
            pltpu.async_copy(x_hbm.at[idxb], rows, sem).wait()
            pltpu.sync_copy(rows, xs_hbm.at[pl.ds(base, GCH)])


def _ffn_body(te_ref, nt_ref, xs_ref, wg_ref, wu_ref, wd_ref, pr_ref, ys_ref):
    t = pl.program_id(0)

    @pl.when(t < nt_ref[0])
    def _():
        xb = xs_ref[...]
        a = jnp.dot(xb, wg_ref[0], preferred_element_type=jnp.float32)
        b = jnp.dot(xb, wu_ref[0], preferred_element_type=jnp.float32)
        h = (a * jax.nn.sigmoid(a)) * b
        y = jnp.dot(h, wd_ref[0], preferred_element_type=jnp.float32)
        ys_ref[...] = y * pr_ref[...]


def _ffn(te, ntv, xs, gate_weights, up_weights, down_weights, pslot):
    def _tm(t, nt):
        return jnp.minimum(t, nt[0] - 1)

    grid_spec = pltpu.PrefetchScalarGridSpec(
        num_scalar_prefetch=2,
        grid=(NT,),
        in_specs=[
            pl.BlockSpec((BM, D), lambda t, te, nt: (_tm(t, nt), 0)),
            pl.BlockSpec((1, D, F), lambda t, te, nt: (te[_tm(t, nt)], 0, 0)),
            pl.BlockSpec((1, D, F), lambda t, te, nt: (te[_tm(t, nt)], 0, 0)),
            pl.BlockSpec((1, F, D), lambda t, te, nt: (te[_tm(t, nt)], 0, 0)),
            pl.BlockSpec((BM, 1), lambda t, te, nt: (_tm(t, nt), 0)),
        ],
        out_specs=pl.BlockSpec((BM, D), lambda t, te, nt: (_tm(t, nt), 0)),
    )
    return pl.pallas_call(
        _ffn_body,
        grid_spec=grid_spec,
        out_shape=jax.ShapeDtypeStruct((PT, D), jnp.float32),
    )(te, ntv, xs, gate_weights, up_weights, down_weights, pslot)


@functools.partial(
    pl.kernel,
    out_type=jax.ShapeDtypeStruct((T, D), jnp.float32),
    mesh=_mesh,
    compiler_params=_CP,
    scratch_types=[
        pltpu.VMEM((T // 32,), jnp.int32),
        pltpu.VMEM((T // 32, D), jnp.float32),
        pltpu.SemaphoreType.DMA,
    ],
)
def _combine(ys_hbm, slot_hbm, out_hbm, idxb, rows, sem):
    c = lax.axis_index("c")
    s = lax.axis_index("s")
    base = (s * 2 + c) * (T // 32)
    pltpu.sync_copy(slot_hbm.at[pl.ds(base, T // 32)], idxb)
    pltpu.async_copy(ys_hbm.at[idxb], rows, sem).wait()
    pltpu.sync_copy(rows, out_hbm.at[pl.ds(base, T // 32)])


def _dispatch(idx, prob):
    cnt, tokscr, pscr = _count(idx, prob)
    return _place(cnt, tokscr, pscr)


def kernel(x, gate_w, gate_weights, up_weights, down_weights):
    idx2, p2 = _router(x, gate_w)
    tok, pslot, slotp, tile, ntv = _dispatch(idx2.reshape(T), p2.reshape(T))
    xs = _gather(x, tok, ntv)
    ys = _ffn(tile, ntv, xs, gate_weights, up_weights, down_weights,
              pslot.reshape(PT, 1))
    return _combine(ys, slotp[:T])
